# baseline (speedup 1.0000x reference)
import functools

import jax
import jax.numpy as jnp
from jax.experimental import pallas as pl
from jax.experimental.pallas import tpu as pltpu

_UPDATE_RATE = 0.5
_AGE_FACTOR = 0.98


def _body(S, B, NS, M,
          hs_ref, mem0_ref,
          win_ref, wval_ref,
          wgx_ref, wgh_ref, wux_ref, wuh_ref, wrx_ref, wrh_ref,
          bin_ref, bval_ref, bg_ref, bu_ref, br_ref,
          out_ref,
          min_scr, xg_scr, xu_scr, xr_scr):
    f32 = jnp.float32

    hs = hs_ref[...]
    m_in_all = jnp.dot(hs, win_ref[...], preferred_element_type=f32) + bin_ref[...]
    vals = jnp.dot(hs, wval_ref[...], preferred_element_type=f32) + bval_ref[...]
    xg_all = jnp.dot(vals, wgx_ref[...], preferred_element_type=f32) + bg_ref[...]
    xu_all = jnp.dot(vals, wux_ref[...], preferred_element_type=f32) + bu_ref[...]
    xr_all = jnp.dot(vals, wrx_ref[...], preferred_element_type=f32) + br_ref[...]
    min_scr[...] = m_in_all.reshape(S, B, M)
    xg_scr[...] = xg_all.reshape(S, B, M)
    xu_scr[...] = xu_all.reshape(S, B, M)
    xr_scr[...] = xr_all.reshape(S, B, M)

    wgh = wgh_ref[...]
    wuh = wuh_ref[...]
    wrh = wrh_ref[...]

    H = 1

    def gru_chunk(memC, xrC, xgC, xuC, ww3C):
        mem2 = memC.reshape(H * NS, M)
        reset = jax.nn.sigmoid(
            jnp.dot(mem2, wrh, preferred_element_type=f32).reshape(H, NS, M)
            + xrC[:, None, :])
        upd = jax.nn.sigmoid(
            jnp.dot(mem2, wgh, preferred_element_type=f32).reshape(H, NS, M)
            + xgC[:, None, :])
        rh = (reset * memC).reshape(H * NS, M)
        cand = jnp.tanh(
            jnp.dot(rh, wuh, preferred_element_type=f32).reshape(H, NS, M)
            + xuC[:, None, :])
        updated = memC + (ww3C * upd) * (cand - memC)
        nsq = jnp.sum(updated * updated, axis=2, keepdims=True)
        return updated * jax.lax.rsqrt(jnp.maximum(nsq, 1e-24))

    def step(t, carry):
        memA, memB, memC, memD, usage, age = carry
        m_in = min_scr[t]
        xg = xg_scr[t]
        xu = xu_scr[t]
        xr = xr_scr[t]

        simA = jnp.sum(memA * m_in[:H, None, :], axis=2)
        simB = jnp.sum(memB * m_in[H:2, None, :], axis=2)
        simC = jnp.sum(memC * m_in[2:3, None, :], axis=2)
        simD = jnp.sum(memD * m_in[3:, None, :], axis=2)
        sim = jnp.concatenate([simA, simB, simC, simD], axis=0)
        scores = usage * 0.2 + age * 0.1 - sim
        w = scores - jnp.max(scores, axis=1, keepdims=True)
        e = jnp.exp(w)
        write_w = e / jnp.sum(e, axis=1, keepdims=True)
        wwm = jnp.where(write_w > 0.01, write_w, jnp.zeros_like(write_w))
        ww3 = (wwm * _UPDATE_RATE)[:, :, None]

        memnA = gru_chunk(memA, xr[:H], xg[:H], xu[:H], ww3[:H])
        memnB = gru_chunk(memB, xr[H:2], xg[H:2], xu[H:2], ww3[H:2])
        memnC = gru_chunk(memC, xr[2:3], xg[2:3], xu[2:3], ww3[2:3])
        memnD = gru_chunk(memD, xr[3:], xg[3:], xu[3:], ww3[3:])

        usage = (usage + wwm) * 0.99
        age = age * _AGE_FACTOR + 1.0
        return memnA, memnB, memnC, memnD, usage, age

    zeros = jnp.zeros((B, NS), dtype=f32)
    memA, memB, memC, memD, _, _ = jax.lax.fori_loop(
        0, S, step,
        (mem0_ref[:1], mem0_ref[1:2], mem0_ref[2:3], mem0_ref[3:], zeros, zeros),
        unroll=4)
    out_ref[:1] = memA
    out_ref[1:2] = memB
    out_ref[2:3] = memC
    out_ref[3:] = memD


@jax.jit
def kernel(hidden_states, memory0, W_in, b_in, W_val, b_val,
           W_gate, b_gate, W_upd, b_upd, W_reset, b_reset):
    B, S, D = hidden_states.shape
    _, NS, M = memory0.shape

    hs = jnp.transpose(hidden_states, (1, 0, 2)).reshape(S * B, D)
    win_t = W_in.T
    wval_t = W_val.T
    wgx, wgh = W_gate[:, :M].T, W_gate[:, M:].T
    wux, wuh = W_upd[:, :M].T, W_upd[:, M:].T
    wrx, wrh = W_reset[:, :M].T, W_reset[:, M:].T

    body = functools.partial(_body, S, B, NS, M)
    out = pl.pallas_call(
        body,
        out_shape=jax.ShapeDtypeStruct((B, NS, M), jnp.float32),
        scratch_shapes=[pltpu.VMEM((S, B, M), jnp.float32)] * 4,
    )(hs, memory0,
      win_t, wval_t, wgx, wgh, wux, wuh, wrx, wrh,
      b_in.reshape(1, M), b_val.reshape(1, M), b_gate.reshape(1, M),
      b_upd.reshape(1, M), b_reset.reshape(1, M))
    return out


# speedup vs baseline: 1.3495x; 1.3495x over previous
import functools

import jax
import jax.numpy as jnp
from jax.experimental import pallas as pl
from jax.experimental.pallas import tpu as pltpu

_UPDATE_RATE = 0.5
_AGE_FACTOR = 0.98


def _body(S, B, NS, M,
          hs_ref, mem0_ref,
          win_ref, wval_ref,
          wgx_ref, wgh_ref, wux_ref, wuh_ref, wrx_ref, wrh_ref,
          bin_ref, bval_ref, bg_ref, bu_ref, br_ref,
          out_ref,
          min_scr, xg_scr, xu_scr, xr_scr):
    f32 = jnp.float32

    hs = hs_ref[...]
    m_in_all = jnp.dot(hs, win_ref[...], preferred_element_type=f32) + bin_ref[...]
    vals = jnp.dot(hs, wval_ref[...], preferred_element_type=f32) + bval_ref[...]
    xg_all = jnp.dot(vals, wgx_ref[...], preferred_element_type=f32) + bg_ref[...]
    xu_all = jnp.dot(vals, wux_ref[...], preferred_element_type=f32) + bu_ref[...]
    xr_all = jnp.dot(vals, wrx_ref[...], preferred_element_type=f32) + br_ref[...]
    min_scr[:S] = m_in_all.reshape(S, B, M)
    min_scr[S:] = jnp.zeros((1, B, M), dtype=f32)
    xg_scr[...] = xg_all.reshape(S, B, M)
    xu_scr[...] = xu_all.reshape(S, B, M)
    xr_scr[...] = xr_all.reshape(S, B, M)

    wgh = wgh_ref[...]
    wuh = wuh_ref[...]
    wrh = wrh_ref[...]

    H = B // 2

    def gru_chunk(memC, xrC, xgC, xuC, ww3C, m_in_nextC):
        mem2 = memC.reshape(H * NS, M)
        reset = jax.nn.sigmoid(
            jnp.dot(mem2, wrh, preferred_element_type=f32).reshape(H, NS, M)
            + xrC[:, None, :])
        upd = jax.nn.sigmoid(
            jnp.dot(mem2, wgh, preferred_element_type=f32).reshape(H, NS, M)
            + xgC[:, None, :])
        rh = (reset * memC).reshape(H * NS, M)
        cand = jnp.tanh(
            jnp.dot(rh, wuh, preferred_element_type=f32).reshape(H, NS, M)
            + xuC[:, None, :])
        updated = memC + (ww3C * upd) * (cand - memC)
        nsq = jnp.sum(updated * updated, axis=2, keepdims=True)
        inv = jax.lax.rsqrt(jnp.maximum(nsq, 1e-24))
        dotn = jnp.sum(updated * m_in_nextC[:, None, :], axis=2, keepdims=True)
        sim_next = (dotn * inv)[:, :, 0]
        return updated * inv, sim_next

    def step(t, carry):
        memA, memB, simA, simB, usage, age = carry
        xg = xg_scr[t]
        xu = xu_scr[t]
        xr = xr_scr[t]
        m_in_next = min_scr[t + 1]

        sim = jnp.concatenate([simA, simB], axis=0)
        scores = usage * 0.2 + age * 0.1 - sim
        w = scores - jnp.max(scores, axis=1, keepdims=True)
        e = jnp.exp(w)
        write_w = e / jnp.sum(e, axis=1, keepdims=True)
        wwm = jnp.where(write_w > 0.01, write_w, jnp.zeros_like(write_w))
        ww3 = (wwm * _UPDATE_RATE)[:, :, None]

        memnA, simnA = gru_chunk(memA, xr[:H], xg[:H], xu[:H], ww3[:H],
                                 m_in_next[:H])
        memnB, simnB = gru_chunk(memB, xr[H:], xg[H:], xu[H:], ww3[H:],
                                 m_in_next[H:])

        usage = (usage + wwm) * 0.99
        age = age * _AGE_FACTOR + 1.0
        return memnA, memnB, simnA, simnB, usage, age

    zeros = jnp.zeros((B, NS), dtype=f32)
    mem0A = mem0_ref[:H]
    mem0B = mem0_ref[H:]
    m_in0 = min_scr[0]
    sim0A = jnp.sum(mem0A * m_in0[:H, None, :], axis=2)
    sim0B = jnp.sum(mem0B * m_in0[H:, None, :], axis=2)
    memA, memB, _, _, _, _ = jax.lax.fori_loop(
        0, S, step, (mem0A, mem0B, sim0A, sim0B, zeros, zeros),
        unroll=4)
    out_ref[:H] = memA
    out_ref[H:] = memB


@jax.jit
def kernel(hidden_states, memory0, W_in, b_in, W_val, b_val,
           W_gate, b_gate, W_upd, b_upd, W_reset, b_reset):
    B, S, D = hidden_states.shape
    _, NS, M = memory0.shape

    hs = jnp.transpose(hidden_states, (1, 0, 2)).reshape(S * B, D)
    win_t = W_in.T
    wval_t = W_val.T
    wgx, wgh = W_gate[:, :M].T, W_gate[:, M:].T
    wux, wuh = W_upd[:, :M].T, W_upd[:, M:].T
    wrx, wrh = W_reset[:, :M].T, W_reset[:, M:].T

    body = functools.partial(_body, S, B, NS, M)
    out = pl.pallas_call(
        body,
        out_shape=jax.ShapeDtypeStruct((B, NS, M), jnp.float32),
        scratch_shapes=[pltpu.VMEM((S + 1, B, M), jnp.float32),
                        pltpu.VMEM((S, B, M), jnp.float32),
                        pltpu.VMEM((S, B, M), jnp.float32),
                        pltpu.VMEM((S, B, M), jnp.float32)],
    )(hs, memory0,
      win_t, wval_t, wgx, wgh, wux, wuh, wrx, wrh,
      b_in.reshape(1, M), b_val.reshape(1, M), b_gate.reshape(1, M),
      b_upd.reshape(1, M), b_reset.reshape(1, M))
    return out


# speedup vs baseline: 1.3807x; 1.0231x over previous
import functools

import jax
import jax.numpy as jnp
from jax.experimental import pallas as pl
from jax.experimental.pallas import tpu as pltpu

_UPDATE_RATE = 0.5
_AGE_FACTOR = 0.98


def _body(S, B, NS, M,
          hs_ref, mem0_ref,
          win_ref, wval_ref,
          wgx_ref, wgh_ref, wux_ref, wuh_ref, wrx_ref, wrh_ref,
          bin_ref, bval_ref, bg_ref, bu_ref, br_ref,
          out_ref,
          min_scr, xg_scr, xu_scr, xr_scr):
    f32 = jnp.float32

    hs = hs_ref[...]
    m_in_all = jnp.dot(hs, win_ref[...], preferred_element_type=f32) + bin_ref[...]
    vals = jnp.dot(hs, wval_ref[...], preferred_element_type=f32) + bval_ref[...]
    xg_all = jnp.dot(vals, wgx_ref[...], preferred_element_type=f32) + bg_ref[...]
    xu_all = jnp.dot(vals, wux_ref[...], preferred_element_type=f32) + bu_ref[...]
    xr_all = jnp.dot(vals, wrx_ref[...], preferred_element_type=f32) + br_ref[...]
    min_scr[:S] = m_in_all.reshape(S, B, M)
    min_scr[S:] = jnp.zeros((1, B, M), dtype=f32)
    xg_scr[...] = xg_all.reshape(S, B, M)
    xu_scr[...] = xu_all.reshape(S, B, M)
    xr_scr[...] = xr_all.reshape(S, B, M)

    wgh = wgh_ref[...]
    wuh = wuh_ref[...]
    wrh = wrh_ref[...]

    H = B // 2

    def gru_chunk(UC, invC, xrC, xgC, xuC, ww3C, m_in_nextC):
        U2 = UC.reshape(H * NS, M)
        r_pre = jnp.dot(U2, wrh, preferred_element_type=f32).reshape(H, NS, M)
        g_pre = jnp.dot(U2, wgh, preferred_element_type=f32).reshape(H, NS, M)
        reset = jax.nn.sigmoid(invC * r_pre + xrC[:, None, :])
        upd = jax.nn.sigmoid(invC * g_pre + xgC[:, None, :])
        memn = UC * invC
        rh = (reset * memn).reshape(H * NS, M)
        cand = jnp.tanh(
            jnp.dot(rh, wuh, preferred_element_type=f32).reshape(H, NS, M)
            + xuC[:, None, :])
        Unew = memn + (ww3C * upd) * (cand - memn)
        nsq = jnp.sum(Unew * Unew, axis=2, keepdims=True)
        invn = jax.lax.rsqrt(jnp.maximum(nsq, 1e-24))
        dotn = jnp.sum(Unew * m_in_nextC[:, None, :], axis=2, keepdims=True)
        sim_next = (dotn * invn)[:, :, 0]
        return Unew, invn, sim_next

    def step(t, carry):
        UA, UB, invA, invB, simA, simB, usage, age = carry
        xg = xg_scr[t]
        xu = xu_scr[t]
        xr = xr_scr[t]
        m_in_next = min_scr[t + 1]

        sim = jnp.concatenate([simA, simB], axis=0)
        scores = usage * 0.2 + age * 0.1 - sim
        w = scores - jnp.max(scores, axis=1, keepdims=True)
        e = jnp.exp(w)
        write_w = e / jnp.sum(e, axis=1, keepdims=True)
        wwm = jnp.where(write_w > 0.01, write_w, jnp.zeros_like(write_w))
        ww3 = (wwm * _UPDATE_RATE)[:, :, None]

        UnA, invnA, simnA = gru_chunk(UA, invA, xr[:H], xg[:H], xu[:H],
                                      ww3[:H], m_in_next[:H])
        UnB, invnB, simnB = gru_chunk(UB, invB, xr[H:], xg[H:], xu[H:],
                                      ww3[H:], m_in_next[H:])

        usage = (usage + wwm) * 0.99
        age = age * _AGE_FACTOR + 1.0
        return UnA, UnB, invnA, invnB, simnA, simnB, usage, age

    zeros = jnp.zeros((B, NS), dtype=f32)
    mem0A = mem0_ref[:H]
    mem0B = mem0_ref[H:]
    m_in0 = min_scr[0]
    sim0A = jnp.sum(mem0A * m_in0[:H, None, :], axis=2)
    sim0B = jnp.sum(mem0B * m_in0[H:, None, :], axis=2)
    ones = jnp.ones((H, NS, 1), dtype=f32)
    UA, UB, invA, invB, _, _, _, _ = jax.lax.fori_loop(
        0, S, step, (mem0A, mem0B, ones, ones, sim0A, sim0B, zeros, zeros),
        unroll=4)
    out_ref[:H] = UA * invA
    out_ref[H:] = UB * invB


@jax.jit
def kernel(hidden_states, memory0, W_in, b_in, W_val, b_val,
           W_gate, b_gate, W_upd, b_upd, W_reset, b_reset):
    B, S, D = hidden_states.shape
    _, NS, M = memory0.shape

    hs = jnp.transpose(hidden_states, (1, 0, 2)).reshape(S * B, D)
    win_t = W_in.T
    wval_t = W_val.T
    wgx, wgh = W_gate[:, :M].T, W_gate[:, M:].T
    wux, wuh = W_upd[:, :M].T, W_upd[:, M:].T
    wrx, wrh = W_reset[:, :M].T, W_reset[:, M:].T

    body = functools.partial(_body, S, B, NS, M)
    out = pl.pallas_call(
        body,
        out_shape=jax.ShapeDtypeStruct((B, NS, M), jnp.float32),
        scratch_shapes=[pltpu.VMEM((S + 1, B, M), jnp.float32),
                        pltpu.VMEM((S, B, M), jnp.float32),
                        pltpu.VMEM((S, B, M), jnp.float32),
                        pltpu.VMEM((S, B, M), jnp.float32)],
    )(hs, memory0,
      win_t, wval_t, wgx, wgh, wux, wuh, wrx, wrh,
      b_in.reshape(1, M), b_val.reshape(1, M), b_gate.reshape(1, M),
      b_upd.reshape(1, M), b_reset.reshape(1, M))
    return out
